# baseline (speedup 1.0000x reference)
import functools

import jax
import jax.numpy as jnp
from jax import lax
from jax.experimental import pallas as pl
from jax.experimental.pallas import tpu as pltpu
from jax.experimental.pallas import tpu_sc as plsc

_COMMIT = 0.25
_LANES = 128
_ROWS_PER_BLOCK = 1024
_SC_CHUNK = 128


def _vq_tc_body(nblocks, kdim, x_ref, w_ref, idx_ref, loss_ref, wt_ref):
    f = pl.program_id(0)
    nb = pl.program_id(1)
    x = x_ref[0]
    w = w_ref[0]
    x2 = x * -2.0
    xsq = jnp.sum(x * x, axis=1, keepdims=True)
    ngrp = kdim // _LANES
    liota = lax.broadcasted_iota(jnp.int32, (1, _LANES), 1)
    minval = None
    kwin = None
    for j in range(ngrp):
        sl = slice(j * _LANES, (j + 1) * _LANES)
        wj = w[:, sl]
        dotsj = jnp.dot(x2, wj, preferred_element_type=jnp.float32)
        wsqj = jnp.sum(wj * wj, axis=0, keepdims=True)
        dj = (xsq + dotsj) + wsqj
        if j == 0:
            minval = dj
            kwin = jnp.broadcast_to(liota, dj.shape)
        else:
            better = dj < minval
            minval = jnp.where(better, dj, minval)
            kwin = jnp.where(better, liota + jnp.int32(j * _LANES), kwin)
    mind = jnp.min(minval, axis=1)
    masked = jnp.where(minval == mind[:, None], kwin, jnp.int32(kdim))
    idx = jnp.min(masked, axis=1)
    idx_ref[0, 0] = idx + f * kdim

    @pl.when(nb == 0)
    def _():
        wt_ref[0] = jnp.swapaxes(w, 0, 1)

    @pl.when(jnp.logical_and(f == 0, nb == 0))
    def _():
        loss_ref[0, 0] = 0.0

    loss_ref[0, 0] += jnp.sum(mind)


def _vq_assign(inputs, W):
    F, N, D = inputs.shape
    K = W.shape[2]
    Nb = _ROWS_PER_BLOCK
    NB = N // Nb
    idx_out, loss_out, wt = pl.pallas_call(
        functools.partial(_vq_tc_body, NB, K),
        grid=(F, NB),
        in_specs=[
            pl.BlockSpec((1, Nb, D), lambda f, nb: (f, nb, 0)),
            pl.BlockSpec((1, D, K), lambda f, nb: (f, 0, 0)),
        ],
        out_specs=[
            pl.BlockSpec((1, 1, Nb), lambda f, nb: (f * NB + nb, 0, 0)),
            pl.BlockSpec((1, 1), lambda f, nb: (0, 0),
                         memory_space=pltpu.SMEM),
            pl.BlockSpec((1, K, D), lambda f, nb: (f, 0, 0)),
        ],
        out_shape=[
            jax.ShapeDtypeStruct((F * NB, 1, Nb), jnp.int32),
            jax.ShapeDtypeStruct((1, 1), jnp.float32),
            jax.ShapeDtypeStruct((F, K, D), jnp.float32),
        ],
    )(inputs, W)
    return idx_out.reshape(F * N), loss_out[0, 0], wt


def _sc_gather(table, idx):
    B = idx.shape[0]
    Dd = table.shape[1]
    info = plsc.get_sparse_core_info()
    nc, ns = info.num_cores, info.num_subcores
    nw = nc * ns
    b_per_w = B // nw
    cb = min(_SC_CHUNK, b_per_w)
    n_chunks = b_per_w // cb
    mesh = plsc.VectorSubcoreMesh(core_axis_name="c", subcore_axis_name="s")

    @functools.partial(
        pl.kernel,
        mesh=mesh,
        out_type=jax.ShapeDtypeStruct((B, Dd), jnp.float32),
        scratch_types=[
            pltpu.VMEM((b_per_w,), jnp.int32),
            pltpu.VMEM((cb, Dd), jnp.float32),
            pltpu.VMEM((cb, Dd), jnp.float32),
            pltpu.SemaphoreType.DMA,
            pltpu.SemaphoreType.DMA,
            pltpu.SemaphoreType.DMA,
            pltpu.SemaphoreType.DMA,
        ],
    )
    def gather_k(table_hbm, idx_hbm, out_hbm, idx_v, buf0, buf1,
                 g0, g1, s0, s1):
        wid = lax.axis_index("s") * nc + lax.axis_index("c")
        base = wid * b_per_w
        pltpu.sync_copy(idx_hbm.at[pl.ds(base, b_per_w)], idx_v)
        bufs = (buf0, buf1)
        gsems = (g0, g1)
        ssems = (s0, s1)
        gd = [None] * n_chunks
        sd = [None] * n_chunks
        gd[0] = pltpu.async_copy(
            table_hbm.at[idx_v.at[pl.ds(0, cb)]], bufs[0], gsems[0])
        for i in range(n_chunks):
            if i + 1 < n_chunks:
                if i + 1 >= 2:
                    sd[i - 1].wait()
                gd[i + 1] = pltpu.async_copy(
                    table_hbm.at[idx_v.at[pl.ds((i + 1) * cb, cb)]],
                    bufs[(i + 1) % 2], gsems[(i + 1) % 2])
            gd[i].wait()
            sd[i] = pltpu.async_copy(
                bufs[i % 2], out_hbm.at[pl.ds(base + i * cb, cb)],
                ssems[i % 2])
        if n_chunks >= 2:
            sd[n_chunks - 2].wait()
        sd[n_chunks - 1].wait()

    return gather_k(table, idx)


def kernel(inputs, W):
    F, N, D = inputs.shape
    K = W.shape[2]
    idx_flat, loss_sum, wt = _vq_assign(inputs, W)
    quantized = _sc_gather(wt.reshape(F * K, D), idx_flat).reshape(F, N, D)
    loss = loss_sum * ((1.0 + _COMMIT) / (F * N * D))
    return quantized, loss


# speedup vs baseline: 1.0630x; 1.0630x over previous
import functools

import jax
import jax.numpy as jnp
from jax import lax
from jax.experimental import pallas as pl
from jax.experimental.pallas import tpu as pltpu
from jax.experimental.pallas import tpu_sc as plsc

_COMMIT = 0.25
_LANES = 128
_ROWS_PER_BLOCK = 2048
_SC_CHUNK = 128


def _vq_tc_body(nblocks, kdim, x_ref, w_ref, idx_ref, loss_ref, wt_ref):
    f = pl.program_id(0)
    nb = pl.program_id(1)
    x = x_ref[0]
    w = w_ref[0]
    dots2 = jnp.dot(x * -2.0, w, preferred_element_type=jnp.float32)
    wsq = jnp.sum(w * w, axis=0, keepdims=True)
    xsq = jnp.sum(x * x, axis=1, keepdims=True)
    ngrp = kdim // _LANES
    liota = lax.broadcasted_iota(jnp.int32, (1, _LANES), 1)
    minval = (xsq + dots2[:, 0:_LANES]) + wsq[:, 0:_LANES]
    kwin = jnp.broadcast_to(liota, minval.shape)
    for j in range(1, ngrp):
        sl = slice(j * _LANES, (j + 1) * _LANES)
        dj = (xsq + dots2[:, sl]) + wsq[:, sl]
        better = dj < minval
        minval = jnp.where(better, dj, minval)
        kwin = jnp.where(better, liota + jnp.int32(j * _LANES), kwin)
    mind = jnp.min(minval, axis=1)
    masked = jnp.where(minval == mind[:, None], kwin, jnp.int32(kdim))
    idx = jnp.min(masked, axis=1)
    idx_ref[0, 0] = idx + f * kdim

    @pl.when(nb == 0)
    def _():
        wt_ref[0] = jnp.swapaxes(w, 0, 1)

    @pl.when(jnp.logical_and(f == 0, nb == 0))
    def _():
        loss_ref[0, 0] = 0.0

    loss_ref[0, 0] += jnp.sum(mind)


def _vq_assign(inputs, W):
    F, N, D = inputs.shape
    K = W.shape[2]
    Nb = _ROWS_PER_BLOCK
    NB = N // Nb
    idx_out, loss_out, wt = pl.pallas_call(
        functools.partial(_vq_tc_body, NB, K),
        grid=(F, NB),
        in_specs=[
            pl.BlockSpec((1, Nb, D), lambda f, nb: (f, nb, 0)),
            pl.BlockSpec((1, D, K), lambda f, nb: (f, 0, 0)),
        ],
        out_specs=[
            pl.BlockSpec((1, 1, Nb), lambda f, nb: (f * NB + nb, 0, 0)),
            pl.BlockSpec((1, 1), lambda f, nb: (0, 0),
                         memory_space=pltpu.SMEM),
            pl.BlockSpec((1, K, D), lambda f, nb: (f, 0, 0)),
        ],
        out_shape=[
            jax.ShapeDtypeStruct((F * NB, 1, Nb), jnp.int32),
            jax.ShapeDtypeStruct((1, 1), jnp.float32),
            jax.ShapeDtypeStruct((F, K, D), jnp.float32),
        ],
    )(inputs, W)
    return idx_out.reshape(F * N), loss_out[0, 0], wt


def _sc_gather(table, idx):
    B = idx.shape[0]
    Dd = table.shape[1]
    info = plsc.get_sparse_core_info()
    nc, ns = info.num_cores, info.num_subcores
    nw = nc * ns
    b_per_w = B // nw
    cb = min(_SC_CHUNK, b_per_w)
    n_chunks = b_per_w // cb
    mesh = plsc.VectorSubcoreMesh(core_axis_name="c", subcore_axis_name="s")

    @functools.partial(
        pl.kernel,
        mesh=mesh,
        out_type=jax.ShapeDtypeStruct((B, Dd), jnp.float32),
        scratch_types=[
            pltpu.VMEM((b_per_w,), jnp.int32),
            pltpu.VMEM((cb, Dd), jnp.float32),
            pltpu.VMEM((cb, Dd), jnp.float32),
            pltpu.SemaphoreType.DMA,
            pltpu.SemaphoreType.DMA,
            pltpu.SemaphoreType.DMA,
            pltpu.SemaphoreType.DMA,
        ],
    )
    def gather_k(table_hbm, idx_hbm, out_hbm, idx_v, buf0, buf1,
                 g0, g1, s0, s1):
        wid = lax.axis_index("s") * nc + lax.axis_index("c")
        base = wid * b_per_w
        pltpu.sync_copy(idx_hbm.at[pl.ds(base, b_per_w)], idx_v)
        bufs = (buf0, buf1)
        gsems = (g0, g1)
        ssems = (s0, s1)
        gd = [None] * n_chunks
        sd = [None] * n_chunks
        gd[0] = pltpu.async_copy(
            table_hbm.at[idx_v.at[pl.ds(0, cb)]], bufs[0], gsems[0])
        for i in range(n_chunks):
            if i + 1 < n_chunks:
                if i + 1 >= 2:
                    sd[i - 1].wait()
                gd[i + 1] = pltpu.async_copy(
                    table_hbm.at[idx_v.at[pl.ds((i + 1) * cb, cb)]],
                    bufs[(i + 1) % 2], gsems[(i + 1) % 2])
            gd[i].wait()
            sd[i] = pltpu.async_copy(
                bufs[i % 2], out_hbm.at[pl.ds(base + i * cb, cb)],
                ssems[i % 2])
        if n_chunks >= 2:
            sd[n_chunks - 2].wait()
        sd[n_chunks - 1].wait()

    return gather_k(table, idx)


def kernel(inputs, W):
    F, N, D = inputs.shape
    K = W.shape[2]
    idx_flat, loss_sum, wt = _vq_assign(inputs, W)
    quantized = _sc_gather(wt.reshape(F * K, D), idx_flat).reshape(F, N, D)
    loss = loss_sum * ((1.0 + _COMMIT) / (F * N * D))
    return quantized, loss
